# baseline (speedup 1.0000x reference)
import functools

import jax
import jax.numpy as jnp
from jax import lax
from jax.experimental import pallas as pl
from jax.experimental.pallas import tpu as pltpu
from jax.experimental.pallas import tpu_sc as plsc

N_NODES = 10000
N_EDGES = 320000
IN_DIM = 128
HID = 128
N_CLS = 16
NEG_SLOPE = 0.2

NC = 2
NS = 16
NW = NC * NS

NPAD = 10240
APAD = 10112
CHUNK = 128
CPT = 80
EPAD = NW * CPT * CHUNK

_MESH = plsc.VectorSubcoreMesh(core_axis_name="c", subcore_axis_name="s")


@functools.partial(
    pl.kernel,
    out_type=jax.ShapeDtypeStruct((NC * NPAD,), jnp.float32),
    mesh=_MESH,
    scratch_types=[
        pltpu.VMEM_SHARED((NPAD,), jnp.float32),
        pltpu.VMEM((NPAD // NS,), jnp.float32),
        pltpu.VMEM((CHUNK,), jnp.float32),
        pltpu.VMEM((CPT, CHUNK), jnp.int32),
        pltpu.SemaphoreType.DMA,
    ],
)
def _deg(dst_hbm, out_hbm, deg_sp, z_v, ones_v, idx_v, sem):
    c = lax.axis_index("c")
    s = lax.axis_index("s")
    zn = NPAD // NS

    for i in range(zn // 16):
        z_v[pl.ds(i * 16, 16)] = jnp.zeros((16,), jnp.float32)
    for i in range(CHUNK // 16):
        ones_v[pl.ds(i * 16, 16)] = jnp.ones((16,), jnp.float32)
    pltpu.sync_copy(z_v, deg_sp.at[pl.ds(s * zn, zn)])
    plsc.subcore_barrier()

    rowbase = (c * NS + s) * CPT
    pltpu.sync_copy(dst_hbm.at[pl.ds(rowbase, CPT)], idx_v)

    def deg_body(j, carry):
        pltpu.async_copy(ones_v, deg_sp.at[idx_v.at[j]], sem, add=True)
        return carry

    lax.fori_loop(0, CPT, deg_body, 0)

    def deg_drain(j, carry):
        pltpu.make_async_copy(ones_v, deg_sp.at[idx_v.at[j]], sem).wait()
        return carry

    lax.fori_loop(0, CPT, deg_drain, 0)
    plsc.subcore_barrier()

    pltpu.sync_copy(deg_sp.at[pl.ds(s * zn, zn)],
                    out_hbm.at[pl.ds(c * NPAD + s * zn, zn)])


def _scale_body(pdeg_ref, x_ref, xs_ref, dinv_ref):
    d = pdeg_ref[0] + pdeg_ref[1] + 1.0
    y = lax.rsqrt(d)
    dinv_ref[...] = y
    xs_ref[...] = x_ref[...] * y


def _scale(pdeg3, x):
    grid = (N_NODES // _ROWS_BLK,)
    return pl.pallas_call(
        _scale_body,
        grid=grid,
        in_specs=[
            pl.BlockSpec((NC, _ROWS_BLK, 1), lambda i: (0, i, 0)),
            pl.BlockSpec((_ROWS_BLK, IN_DIM), lambda i: (i, 0)),
        ],
        out_specs=[
            pl.BlockSpec((_ROWS_BLK, IN_DIM), lambda i: (i, 0)),
            pl.BlockSpec((_ROWS_BLK, 1), lambda i: (i, 0)),
        ],
        out_shape=[
            jax.ShapeDtypeStruct((N_NODES, IN_DIM), jnp.float32),
            jax.ShapeDtypeStruct((N_NODES, 1), jnp.float32),
        ],
    )(pdeg3, x)


def _make_agg(dim, acc_w):
    zrows = APAD // NS
    hch = CPT // 2
    nbuf = 2 if acc_w == dim else 4

    @functools.partial(
        pl.kernel,
        out_type=jax.ShapeDtypeStruct((NC, N_NODES, acc_w), jnp.float32),
        mesh=_MESH,
        scratch_types=[
            pltpu.VMEM_SHARED((APAD, acc_w), jnp.float32),
            pltpu.VMEM((hch, CHUNK), jnp.int32),
            pltpu.VMEM((hch, CHUNK), jnp.int32),
            pltpu.VMEM((CHUNK if acc_w != dim else 8, acc_w), jnp.float32),
            [pltpu.VMEM((CHUNK, dim), jnp.float32) for _ in range(nbuf)],
            [pltpu.SemaphoreType.DMA for _ in range(nbuf)],
        ],
    )
    def agg(tab_hbm, src_hbm, dst_hbm, out_hbm, acc_sp, sidx_v, didx_v,
            cmp_v, rows_v, sems):
        c = lax.axis_index("c")
        s = lax.axis_index("s")

        zsrc = cmp_v if acc_w != dim else rows_v[0]

        def zrow(r, carry):
            for cg in range(acc_w // 16):
                zsrc[r, pl.ds(cg * 16, 16)] = jnp.zeros((16,), jnp.float32)
            return carry

        lax.fori_loop(0, CHUNK, zrow, 0)
        for k in range(zrows // CHUNK):
            pltpu.sync_copy(zsrc, acc_sp.at[pl.ds(s * zrows + k * CHUNK, CHUNK)])
        pltpu.sync_copy(zsrc.at[pl.ds(0, zrows % CHUNK)],
                        acc_sp.at[pl.ds(s * zrows + (zrows // CHUNK) * CHUNK,
                                        zrows % CHUNK)])
        plsc.subcore_barrier()

        def gather_chunk(j, b):
            pltpu.async_copy(tab_hbm.at[sidx_v.at[j]], rows_v[b], sems[b])

        def wait_chunk(j, b):
            pltpu.make_async_copy(
                tab_hbm.at[sidx_v.at[j]], rows_v[b], sems[b]).wait()

        def scatter_chunk(j, buf):
            if acc_w == dim:
                pltpu.sync_copy(buf, acc_sp.at[didx_v.at[j]], add=True)
            else:
                def crow(r, carry):
                    for cg in range(acc_w // 16):
                        cmp_v[r, pl.ds(cg * 16, 16)] = buf[r, pl.ds(cg * 16, 16)]
                    return carry

                lax.fori_loop(0, CHUNK, crow, 0)
                pltpu.sync_copy(cmp_v, acc_sp.at[didx_v.at[j]], add=True)

        for h in range(2):
            rowbase = (c * NS + s) * CPT + h * hch
            pltpu.sync_copy(src_hbm.at[pl.ds(rowbase, hch)], sidx_v)
            pltpu.sync_copy(dst_hbm.at[pl.ds(rowbase, hch)], didx_v)

            for b in range(nbuf):
                gather_chunk(b, b)

            def eloop(t, carry):
                for b in range(nbuf):
                    j = nbuf * t + b
                    wait_chunk(j, b)
                    scatter_chunk(j, rows_v[b])

                    @pl.when(j + nbuf < hch)
                    def _():
                        gather_chunk(j + nbuf, b)

                return carry

            lax.fori_loop(0, hch // nbuf, eloop, 0)
            for j in range((hch // nbuf) * nbuf, hch):
                b = j % nbuf
                wait_chunk(j, b)
                scatter_chunk(j, rows_v[b])
        plsc.subcore_barrier()

        @pl.when(s < NS - 1)
        def _():
            pltpu.sync_copy(acc_sp.at[pl.ds(s * zrows, zrows)],
                            out_hbm.at[c, pl.ds(s * zrows, zrows)])

        @pl.when(s == NS - 1)
        def _():
            pltpu.sync_copy(acc_sp.at[pl.ds(9480, 520)],
                            out_hbm.at[c, pl.ds(9480, 520)])

    return agg


_agg128 = _make_agg(IN_DIM, IN_DIM)


_ROWS_BLK = 1000


def _mlp_body(pp_ref, xs_ref, dinv_ref, w1_ref, b1_ref, w2_ref, gs_ref):
    z = (pp_ref[0] + pp_ref[1] + xs_ref[...]) * dinv_ref[...]
    h = jnp.dot(z, w1_ref[...], preferred_element_type=jnp.float32) + b1_ref[...]
    h = jnp.where(h >= 0, h, NEG_SLOPE * h)
    g = jnp.dot(h, w2_ref[...], preferred_element_type=jnp.float32)
    gs_ref[...] = g * dinv_ref[...]


def _mlp(pp, xs, dinv2, W1, b1r, W2p):
    grid = (N_NODES // _ROWS_BLK,)
    return pl.pallas_call(
        _mlp_body,
        grid=grid,
        in_specs=[
            pl.BlockSpec((NC, _ROWS_BLK, IN_DIM), lambda i: (0, i, 0)),
            pl.BlockSpec((_ROWS_BLK, IN_DIM), lambda i: (i, 0)),
            pl.BlockSpec((_ROWS_BLK, 1), lambda i: (i, 0)),
            pl.BlockSpec((IN_DIM, HID), lambda i: (0, 0)),
            pl.BlockSpec((1, HID), lambda i: (0, 0)),
            pl.BlockSpec((HID, IN_DIM), lambda i: (0, 0)),
        ],
        out_specs=pl.BlockSpec((_ROWS_BLK, IN_DIM), lambda i: (i, 0)),
        out_shape=jax.ShapeDtypeStruct((N_NODES, IN_DIM), jnp.float32),
    )(pp, xs, dinv2, W1, b1r, W2p)


def _fin_body(qq_ref, gs_ref, dinv_ref, b2_ref, out_ref):
    q = qq_ref[0, :, :N_CLS] + qq_ref[1, :, :N_CLS] + gs_ref[:, :N_CLS]
    l = q * dinv_ref[...] + b2_ref[...]
    m = jnp.max(l, axis=1, keepdims=True)
    e = jnp.exp(l - m)
    out_ref[...] = e / jnp.sum(e, axis=1, keepdims=True)


def _finish(qq, gs, dinv2, b2r):
    grid = (N_NODES // _ROWS_BLK,)
    return pl.pallas_call(
        _fin_body,
        grid=grid,
        in_specs=[
            pl.BlockSpec((NC, _ROWS_BLK, IN_DIM), lambda i: (0, i, 0)),
            pl.BlockSpec((_ROWS_BLK, IN_DIM), lambda i: (i, 0)),
            pl.BlockSpec((_ROWS_BLK, 1), lambda i: (i, 0)),
            pl.BlockSpec((1, N_CLS), lambda i: (0, 0)),
        ],
        out_specs=pl.BlockSpec((_ROWS_BLK, N_CLS), lambda i: (i, 0)),
        out_shape=jax.ShapeDtypeStruct((N_NODES, N_CLS), jnp.float32),
    )(qq, gs, dinv2, b2r)


def kernel(x, edge_index, W1, b1, W2, b2):
    src = edge_index[0].astype(jnp.int32)
    dst = edge_index[1].astype(jnp.int32)
    npad_e = EPAD - N_EDGES
    iota = jnp.arange(npad_e, dtype=jnp.int32)
    srcp = jnp.concatenate([src, iota % N_NODES]).reshape(EPAD // CHUNK, CHUNK)
    dstp = jnp.concatenate(
        [dst, N_NODES + iota % (APAD - N_NODES)]).reshape(EPAD // CHUNK, CHUNK)
    pdeg = _deg(dstp).reshape(NC, NPAD)[:, :N_NODES]
    xs, dinv2 = _scale(pdeg.reshape(NC, N_NODES, 1), x)
    pp = _agg128(xs, srcp, dstp)
    W2p = jnp.concatenate(
        [W2, jnp.zeros((HID, IN_DIM - N_CLS), jnp.float32)], axis=1)
    gs = _mlp(pp, xs, dinv2, W1, b1.reshape(1, HID), W2p)
    qq = _agg128(gs, srcp, dstp)
    return _finish(qq, gs, dinv2, b2.reshape(1, N_CLS))


# speedup vs baseline: 1.0046x; 1.0046x over previous
import functools

import jax
import jax.numpy as jnp
from jax import lax
from jax.experimental import pallas as pl
from jax.experimental.pallas import tpu as pltpu
from jax.experimental.pallas import tpu_sc as plsc

N_NODES = 10000
N_EDGES = 320000
IN_DIM = 128
HID = 128
N_CLS = 16
NEG_SLOPE = 0.2

NC = 2
NS = 16
NW = NC * NS

NPAD = 10240
APAD = 10112
CHUNK = 128
CPT = 80
EPAD = NW * CPT * CHUNK

_MESH = plsc.VectorSubcoreMesh(core_axis_name="c", subcore_axis_name="s")


@functools.partial(
    pl.kernel,
    out_type=jax.ShapeDtypeStruct((NC * NPAD,), jnp.float32),
    mesh=_MESH,
    scratch_types=[
        pltpu.VMEM_SHARED((NPAD,), jnp.float32),
        pltpu.VMEM((NPAD // NS,), jnp.float32),
        pltpu.VMEM((CHUNK,), jnp.float32),
        pltpu.VMEM((CPT, CHUNK), jnp.int32),
        pltpu.SemaphoreType.DMA,
    ],
)
def _deg(dst_hbm, out_hbm, deg_sp, z_v, ones_v, idx_v, sem):
    c = lax.axis_index("c")
    s = lax.axis_index("s")
    zn = NPAD // NS

    for i in range(zn // 16):
        z_v[pl.ds(i * 16, 16)] = jnp.zeros((16,), jnp.float32)
    for i in range(CHUNK // 16):
        ones_v[pl.ds(i * 16, 16)] = jnp.ones((16,), jnp.float32)
    pltpu.sync_copy(z_v, deg_sp.at[pl.ds(s * zn, zn)])
    plsc.subcore_barrier()

    rowbase = (c * NS + s) * CPT
    pltpu.sync_copy(dst_hbm.at[pl.ds(rowbase, CPT)], idx_v)

    def deg_body(j, carry):
        pltpu.async_copy(ones_v, deg_sp.at[idx_v.at[j]], sem, add=True)
        return carry

    lax.fori_loop(0, CPT, deg_body, 0)

    def deg_drain(j, carry):
        pltpu.make_async_copy(ones_v, deg_sp.at[idx_v.at[j]], sem).wait()
        return carry

    lax.fori_loop(0, CPT, deg_drain, 0)
    plsc.subcore_barrier()

    pltpu.sync_copy(deg_sp.at[pl.ds(s * zn, zn)],
                    out_hbm.at[pl.ds(c * NPAD + s * zn, zn)])


def _scale_body(pdeg_ref, x_ref, xs_ref, dinv_ref):
    d = pdeg_ref[0] + pdeg_ref[1] + 1.0
    y = lax.rsqrt(d)
    dinv_ref[...] = y
    xs_ref[...] = x_ref[...] * y


def _scale(pdeg3, x):
    grid = (N_NODES // _ROWS_BLK,)
    return pl.pallas_call(
        _scale_body,
        grid=grid,
        in_specs=[
            pl.BlockSpec((NC, _ROWS_BLK, 1), lambda i: (0, i, 0)),
            pl.BlockSpec((_ROWS_BLK, IN_DIM), lambda i: (i, 0)),
        ],
        out_specs=[
            pl.BlockSpec((_ROWS_BLK, IN_DIM), lambda i: (i, 0)),
            pl.BlockSpec((_ROWS_BLK, 1), lambda i: (i, 0)),
        ],
        out_shape=[
            jax.ShapeDtypeStruct((N_NODES, IN_DIM), jnp.float32),
            jax.ShapeDtypeStruct((N_NODES, 1), jnp.float32),
        ],
    )(pdeg3, x)


def _make_agg(dim):
    zrows = APAD // NS
    hch = CPT // 2
    nbuf = 2

    @functools.partial(
        pl.kernel,
        out_type=jax.ShapeDtypeStruct((NC, N_NODES, dim), jnp.float32),
        mesh=_MESH,
        scratch_types=[
            pltpu.VMEM_SHARED((APAD, dim), jnp.float32),
            pltpu.VMEM((hch, CHUNK), jnp.int32),
            pltpu.VMEM((hch, CHUNK), jnp.int32),
            [pltpu.VMEM((CHUNK, dim), jnp.float32) for _ in range(nbuf)],
            [pltpu.SemaphoreType.DMA for _ in range(nbuf)],
        ],
    )
    def agg(tab_hbm, src_hbm, dst_hbm, out_hbm, acc_sp, sidx_v, didx_v,
            rows_v, sems):
        c = lax.axis_index("c")
        s = lax.axis_index("s")

        zsrc = rows_v[0]

        def zrow(r, carry):
            for cg in range(dim // 16):
                zsrc[r, pl.ds(cg * 16, 16)] = jnp.zeros((16,), jnp.float32)
            return carry

        lax.fori_loop(0, CHUNK, zrow, 0)
        for k in range(zrows // CHUNK):
            pltpu.sync_copy(zsrc, acc_sp.at[pl.ds(s * zrows + k * CHUNK, CHUNK)])
        pltpu.sync_copy(zsrc.at[pl.ds(0, zrows % CHUNK)],
                        acc_sp.at[pl.ds(s * zrows + (zrows // CHUNK) * CHUNK,
                                        zrows % CHUNK)])
        plsc.subcore_barrier()

        def gather_chunk(j, b):
            pltpu.async_copy(tab_hbm.at[sidx_v.at[j]], rows_v[b], sems[b])

        def wait_chunk(j, b):
            pltpu.make_async_copy(
                tab_hbm.at[sidx_v.at[j]], rows_v[b], sems[b]).wait()

        def scatter_chunk(j, buf):
            pltpu.sync_copy(buf, acc_sp.at[didx_v.at[j]], add=True)

        for h in range(2):
            rowbase = (c * NS + s) * CPT + h * hch
            pltpu.sync_copy(src_hbm.at[pl.ds(rowbase, hch)], sidx_v)
            pltpu.sync_copy(dst_hbm.at[pl.ds(rowbase, hch)], didx_v)

            for b in range(nbuf):
                gather_chunk(b, b)

            def eloop(t, carry):
                for b in range(nbuf):
                    j = nbuf * t + b
                    wait_chunk(j, b)
                    scatter_chunk(j, rows_v[b])

                    @pl.when(j + nbuf < hch)
                    def _():
                        gather_chunk(j + nbuf, b)

                return carry

            lax.fori_loop(0, hch // nbuf, eloop, 0)
            for j in range((hch // nbuf) * nbuf, hch):
                b = j % nbuf
                wait_chunk(j, b)
                scatter_chunk(j, rows_v[b])
        plsc.subcore_barrier()

        @pl.when(s < NS - 1)
        def _():
            pltpu.sync_copy(acc_sp.at[pl.ds(s * zrows, zrows)],
                            out_hbm.at[c, pl.ds(s * zrows, zrows)])

        @pl.when(s == NS - 1)
        def _():
            pltpu.sync_copy(acc_sp.at[pl.ds(9480, 520)],
                            out_hbm.at[c, pl.ds(9480, 520)])

    return agg


_agg128 = _make_agg(IN_DIM)


_ROWS_BLK = 1000


def _mlp_body(pp_ref, xs_ref, dinv_ref, w1_ref, b1_ref, w2_ref, gs_ref):
    z = (pp_ref[0] + pp_ref[1] + xs_ref[...]) * dinv_ref[...]
    h = jnp.dot(z, w1_ref[...], preferred_element_type=jnp.float32) + b1_ref[...]
    h = jnp.where(h >= 0, h, NEG_SLOPE * h)
    g = jnp.dot(h, w2_ref[...], preferred_element_type=jnp.float32)
    gs_ref[...] = g * dinv_ref[...]


def _mlp(pp, xs, dinv2, W1, b1r, W2p):
    grid = (N_NODES // _ROWS_BLK,)
    return pl.pallas_call(
        _mlp_body,
        grid=grid,
        in_specs=[
            pl.BlockSpec((NC, _ROWS_BLK, IN_DIM), lambda i: (0, i, 0)),
            pl.BlockSpec((_ROWS_BLK, IN_DIM), lambda i: (i, 0)),
            pl.BlockSpec((_ROWS_BLK, 1), lambda i: (i, 0)),
            pl.BlockSpec((IN_DIM, HID), lambda i: (0, 0)),
            pl.BlockSpec((1, HID), lambda i: (0, 0)),
            pl.BlockSpec((HID, IN_DIM), lambda i: (0, 0)),
        ],
        out_specs=pl.BlockSpec((_ROWS_BLK, IN_DIM), lambda i: (i, 0)),
        out_shape=jax.ShapeDtypeStruct((N_NODES, IN_DIM), jnp.float32),
    )(pp, xs, dinv2, W1, b1r, W2p)


def _fin_body(qq_ref, gs_ref, dinv_ref, b2_ref, out_ref):
    q = qq_ref[0, :, :N_CLS] + qq_ref[1, :, :N_CLS] + gs_ref[:, :N_CLS]
    l = q * dinv_ref[...] + b2_ref[...]
    m = jnp.max(l, axis=1, keepdims=True)
    e = jnp.exp(l - m)
    out_ref[...] = e / jnp.sum(e, axis=1, keepdims=True)


def _finish(qq, gs, dinv2, b2r):
    grid = (N_NODES // _ROWS_BLK,)
    return pl.pallas_call(
        _fin_body,
        grid=grid,
        in_specs=[
            pl.BlockSpec((NC, _ROWS_BLK, IN_DIM), lambda i: (0, i, 0)),
            pl.BlockSpec((_ROWS_BLK, IN_DIM), lambda i: (i, 0)),
            pl.BlockSpec((_ROWS_BLK, 1), lambda i: (i, 0)),
            pl.BlockSpec((1, N_CLS), lambda i: (0, 0)),
        ],
        out_specs=pl.BlockSpec((_ROWS_BLK, N_CLS), lambda i: (i, 0)),
        out_shape=jax.ShapeDtypeStruct((N_NODES, N_CLS), jnp.float32),
    )(qq, gs, dinv2, b2r)


def kernel(x, edge_index, W1, b1, W2, b2):
    src = edge_index[0].astype(jnp.int32)
    dst = edge_index[1].astype(jnp.int32)
    npad_e = EPAD - N_EDGES
    iota = jnp.arange(npad_e, dtype=jnp.int32)
    srcp = jnp.concatenate([src, iota % N_NODES]).reshape(EPAD // CHUNK, CHUNK)
    dstp = jnp.concatenate(
        [dst, N_NODES + iota % (APAD - N_NODES)]).reshape(EPAD // CHUNK, CHUNK)
    pdeg = _deg(dstp).reshape(NC, NPAD)[:, :N_NODES]
    xs, dinv2 = _scale(pdeg.reshape(NC, N_NODES, 1), x)
    pp = _agg128(xs, srcp, dstp)
    W2p = jnp.concatenate(
        [W2, jnp.zeros((HID, IN_DIM - N_CLS), jnp.float32)], axis=1)
    gs = _mlp(pp, xs, dinv2, W1, b1.reshape(1, HID), W2p)
    qq = _agg128(gs, srcp, dstp)
    return _finish(qq, gs, dinv2, b2.reshape(1, N_CLS))
